# baseline (speedup 1.0000x reference)
import functools

import jax
import jax.numpy as jnp
from jax import lax
from jax.experimental import pallas as pl
from jax.experimental.pallas import tpu as pltpu
from jax.experimental.pallas import tpu_sc as plsc

_N = 10000
_E = 320000
_CH = 128
_K = 80
_K0 = 64
_K1 = 96
_NW = 32
_EPAD = _NW * _K * _CH
_ROWS = _EPAD // 128
_HALF = _N + 48
_AGG1 = 2 * _HALF
_AGG2 = _N + 112


def _tc_a_body(x_ref, wsst_ref, wsts_ref, wnst_ref, wnts_ref,
               ei_ref, rev_ref, ys_ref, yn_ref, g1_ref, s1_ref, g2_ref):
    x = x_ref[...]
    ys_ref[0:_N, 0:16] = jnp.dot(x, wsst_ref[...],
                                 preferred_element_type=jnp.float32)
    ys_ref[0:_N, 16:32] = jnp.dot(x, wsts_ref[...],
                                  preferred_element_type=jnp.float32)
    yn_ref[0:_N, 0:16] = jnp.dot(x, wnst_ref[...],
                                 preferred_element_type=jnp.float32)
    yn_ref[0:_N, 16:32] = jnp.dot(x, wnts_ref[...],
                                  preferred_element_type=jnp.float32)
    src = ei_ref[0]
    dst = ei_ref[1]
    rev = rev_ref[...]
    g1_ref[...] = src * 8 + rev
    s1_ref[...] = dst + _HALF * rev
    g2_ref[...] = src * 8 + 1


def _tc_a(x, wsst, wsts, wnst, wnts, eip, revp):
    return pl.pallas_call(
        _tc_a_body,
        out_shape=[
            jax.ShapeDtypeStruct((_N, 128), jnp.float32),
            jax.ShapeDtypeStruct((_N, 128), jnp.float32),
            jax.ShapeDtypeStruct((_ROWS, 128), jnp.int32),
            jax.ShapeDtypeStruct((_ROWS, 128), jnp.int32),
            jax.ShapeDtypeStruct((_ROWS, 128), jnp.int32),
        ],
    )(x, wsst, wsts, wnst, wnts, eip, revp)


def _tc_c_body(ys_ref, a_ref, bst_ref, bts_ref,
               ws2a_ref, ws2b_ref, wn2a_ref, wn2b_ref, z_ref):
    a_st = a_ref[0:_N, 0:16] + a_ref[0:_N, 16:32]
    a_ts = (a_ref[_HALF:_HALF + _N, 0:16]
            + a_ref[_HALF:_HALF + _N, 16:32])
    h_st = jnp.maximum(ys_ref[0:_N, 0:16] + a_st + bst_ref[...], 0.0)
    h_ts = jnp.maximum(ys_ref[0:_N, 16:32] + a_ts + bts_ref[...], 0.0)
    z_ref[0:_N, 0:16] = (
        jnp.dot(h_st, ws2a_ref[...], preferred_element_type=jnp.float32)
        + jnp.dot(h_ts, ws2b_ref[...], preferred_element_type=jnp.float32))
    z_ref[0:_N, 16:32] = (
        jnp.dot(h_st, wn2a_ref[...], preferred_element_type=jnp.float32)
        + jnp.dot(h_ts, wn2b_ref[...], preferred_element_type=jnp.float32))


def _tc_c(ys, agg1, bst, bts, ws2a, ws2b, wn2a, wn2b):
    return pl.pallas_call(
        _tc_c_body,
        out_shape=jax.ShapeDtypeStruct((_N, 128), jnp.float32),
    )(ys, agg1, bst, bts, ws2a, ws2b, wn2a, wn2b)


def _tc_e_body(z_ref, a_ref, b_ref, out_ref):
    h = (z_ref[0:_N, 0:16] + a_ref[0:_N, 0:16] + a_ref[0:_N, 16:32]
         + b_ref[...])
    m = jnp.max(h, axis=1, keepdims=True)
    e = jnp.exp(h - m)
    lse = m + jnp.log(jnp.sum(e, axis=1, keepdims=True))
    out_ref[...] = h - lse


def _tc_e(z, agg2, b2):
    return pl.pallas_call(
        _tc_e_body,
        out_shape=jax.ShapeDtypeStruct((_N, 16), jnp.float32),
    )(z, agg2, b2)


def _sc_segsum(table, gidx, sidx, zeros, agg_rows):
    rpt = agg_rows // 16
    kmax = max(_K0, _K1)
    mesh = plsc.VectorSubcoreMesh(core_axis_name="c", subcore_axis_name="s")

    @functools.partial(
        pl.kernel,
        out_type=jax.ShapeDtypeStruct((agg_rows, 128), jnp.float32),
        mesh=mesh,
        scratch_types=[
            pltpu.VMEM((kmax, _CH), jnp.int32),
            pltpu.VMEM((kmax, _CH), jnp.int32),
            pltpu.VMEM((_CH, 16), jnp.float32),
            pltpu.VMEM((_CH, 16), jnp.float32),
            pltpu.VMEM_SHARED((agg_rows, 16), jnp.float32),
            pltpu.SemaphoreType.DMA,
            pltpu.SemaphoreType.DMA,
        ],
        compiler_params=pltpu.CompilerParams(use_tc_tiling_on_sc=False),
    )
    def k(table_hbm, gidx_hbm, sidx_hbm, zeros_hbm, out_hbm,
          gidx_v, sidx_v, v0, v1, agg_sh, sem0, sem1):
        c = lax.axis_index("c")
        s = lax.axis_index("s")
        kc = _K0 + c * (_K1 - _K0)
        base = c * 16 * _K0 + s * kc
        pltpu.sync_copy(gidx_hbm.at[pl.ds(base, kmax)], gidx_v)
        pltpu.sync_copy(sidx_hbm.at[pl.ds(base, kmax)], sidx_v)
        pltpu.sync_copy(zeros_hbm.at[pl.ds(s * rpt, rpt)],
                        agg_sh.at[pl.ds(s * rpt, rpt)])
        plsc.subcore_barrier()

        def start(j, buf, sem):
            pltpu.async_copy(table_hbm.at[gidx_v.at[j]], buf, sem)

        def finish(j, buf, sem):
            pltpu.make_async_copy(table_hbm.at[gidx_v.at[j]], buf, sem).wait()
            pltpu.sync_copy(buf, agg_sh.at[sidx_v.at[j]], add=True)

        start(0, v0, sem0)
        start(1, v1, sem1)

        def body(i, carry):
            j = i * 2
            finish(j, v0, sem0)
            start(j + 2, v0, sem0)
            finish(j + 1, v1, sem1)
            start(j + 3, v1, sem1)
            return carry

        lax.fori_loop(0, kc // 2 - 1, body, 0)
        finish(kc - 2, v0, sem0)
        finish(kc - 1, v1, sem1)
        plsc.subcore_barrier()
        pltpu.sync_copy(agg_sh.at[pl.ds(s * rpt, rpt)],
                        out_hbm.at[pl.ds(s * rpt, rpt), pl.ds(c * 16, 16)])

    return k(table, gidx, sidx, zeros)


def kernel(x, edge_index, is_reversed, Ws_st1, Wn_st1, b_st1,
           Ws_ts1, Wn_ts1, b_ts1, Ws_2, Wn_2, b_2):
    rev = is_reversed.astype(jnp.int32)
    pad = _EPAD - _E
    ei_pad = jnp.broadcast_to(jnp.array([[0], [_N]], jnp.int32), (2, pad))
    eip = jnp.concatenate([edge_index, ei_pad], axis=1).reshape(2, _ROWS, 128)
    revp = jnp.pad(rev, (0, pad)).reshape(_ROWS, 128)

    ys, yn, g1, s1, g2 = _tc_a(x, Ws_st1, Ws_ts1, Wn_st1, Wn_ts1, eip, revp)

    agg1 = _sc_segsum(yn.reshape(8 * _N, 16), g1, s1,
                      jnp.zeros((_AGG1, 16), jnp.float32), _AGG1)
    z = _tc_c(ys, agg1,
              b_st1.reshape(1, 16), b_ts1.reshape(1, 16),
              Ws_2[0:16], Ws_2[16:32], Wn_2[0:16], Wn_2[16:32])

    agg2 = _sc_segsum(z.reshape(8 * _N, 16), g2, eip[1],
                      jnp.zeros((_AGG2, 16), jnp.float32), _AGG2)
    return _tc_e(z, agg2, b_2.reshape(1, 16))


# speedup vs baseline: 1.1347x; 1.1347x over previous
import functools

import jax
import jax.numpy as jnp
from jax import lax
from jax.experimental import pallas as pl
from jax.experimental.pallas import tpu as pltpu
from jax.experimental.pallas import tpu_sc as plsc

_N = 10000
_E = 320000
_CH = 128
_K = 80
_K0 = 96
_K1 = 64
_NW = 32
_EPAD = _NW * _K * _CH
_ROWS = _EPAD // 128
_HALF = _N + 48
_AGG1 = 2 * _HALF
_AGG2 = _N + 112


def _tc_a_body(x_ref, wsst_ref, wsts_ref, wnst_ref, wnts_ref,
               ei_ref, rev_ref, ys_ref, yn_ref, g1_ref, s1_ref, g2_ref):
    x = x_ref[...]
    ys_ref[0:_N, 0:16] = jnp.dot(x, wsst_ref[...],
                                 preferred_element_type=jnp.float32)
    ys_ref[0:_N, 16:32] = jnp.dot(x, wsts_ref[...],
                                  preferred_element_type=jnp.float32)
    yn_ref[0:_N, 0:16] = jnp.dot(x, wnst_ref[...],
                                 preferred_element_type=jnp.float32)
    yn_ref[0:_N, 16:32] = jnp.dot(x, wnts_ref[...],
                                  preferred_element_type=jnp.float32)
    src = ei_ref[0]
    dst = ei_ref[1]
    rev = rev_ref[...]
    g1_ref[...] = src * 8 + rev
    s1_ref[...] = dst + _HALF * rev
    g2_ref[...] = src * 8 + 1


def _tc_a(x, wsst, wsts, wnst, wnts, eip, revp):
    return pl.pallas_call(
        _tc_a_body,
        out_shape=[
            jax.ShapeDtypeStruct((_N, 128), jnp.float32),
            jax.ShapeDtypeStruct((_N, 128), jnp.float32),
            jax.ShapeDtypeStruct((_ROWS, 128), jnp.int32),
            jax.ShapeDtypeStruct((_ROWS, 128), jnp.int32),
            jax.ShapeDtypeStruct((_ROWS, 128), jnp.int32),
        ],
    )(x, wsst, wsts, wnst, wnts, eip, revp)


def _tc_c_body(ys_ref, a_ref, bst_ref, bts_ref,
               ws2a_ref, ws2b_ref, wn2a_ref, wn2b_ref, z_ref):
    a_st = a_ref[0:_N, 0:16] + a_ref[0:_N, 16:32]
    a_ts = (a_ref[_HALF:_HALF + _N, 0:16]
            + a_ref[_HALF:_HALF + _N, 16:32])
    h_st = jnp.maximum(ys_ref[0:_N, 0:16] + a_st + bst_ref[...], 0.0)
    h_ts = jnp.maximum(ys_ref[0:_N, 16:32] + a_ts + bts_ref[...], 0.0)
    z_ref[0:_N, 0:16] = (
        jnp.dot(h_st, ws2a_ref[...], preferred_element_type=jnp.float32)
        + jnp.dot(h_ts, ws2b_ref[...], preferred_element_type=jnp.float32))
    z_ref[0:_N, 16:32] = (
        jnp.dot(h_st, wn2a_ref[...], preferred_element_type=jnp.float32)
        + jnp.dot(h_ts, wn2b_ref[...], preferred_element_type=jnp.float32))


def _tc_c(ys, agg1, bst, bts, ws2a, ws2b, wn2a, wn2b):
    return pl.pallas_call(
        _tc_c_body,
        out_shape=jax.ShapeDtypeStruct((_N, 128), jnp.float32),
    )(ys, agg1, bst, bts, ws2a, ws2b, wn2a, wn2b)


def _tc_e_body(z_ref, a_ref, b_ref, out_ref):
    h = (z_ref[0:_N, 0:16] + a_ref[0:_N, 0:16] + a_ref[0:_N, 16:32]
         + b_ref[...])
    m = jnp.max(h, axis=1, keepdims=True)
    e = jnp.exp(h - m)
    lse = m + jnp.log(jnp.sum(e, axis=1, keepdims=True))
    out_ref[...] = h - lse


def _tc_e(z, agg2, b2):
    return pl.pallas_call(
        _tc_e_body,
        out_shape=jax.ShapeDtypeStruct((_N, 16), jnp.float32),
    )(z, agg2, b2)


def _sc_segsum(table, gidx, sidx, zeros, agg_rows):
    rpt = agg_rows // 16
    kmax = max(_K0, _K1)
    mesh = plsc.VectorSubcoreMesh(core_axis_name="c", subcore_axis_name="s")

    @functools.partial(
        pl.kernel,
        out_type=jax.ShapeDtypeStruct((agg_rows, 128), jnp.float32),
        mesh=mesh,
        scratch_types=[
            pltpu.VMEM((kmax, _CH), jnp.int32),
            pltpu.VMEM((kmax, _CH), jnp.int32),
            pltpu.VMEM((_CH, 16), jnp.float32),
            pltpu.VMEM((_CH, 16), jnp.float32),
            pltpu.VMEM_SHARED((agg_rows, 16), jnp.float32),
            pltpu.SemaphoreType.DMA,
            pltpu.SemaphoreType.DMA,
        ],
        compiler_params=pltpu.CompilerParams(use_tc_tiling_on_sc=False),
    )
    def k(table_hbm, gidx_hbm, sidx_hbm, zeros_hbm, out_hbm,
          gidx_v, sidx_v, v0, v1, agg_sh, sem0, sem1):
        c = lax.axis_index("c")
        s = lax.axis_index("s")
        kc = _K0 + c * (_K1 - _K0)
        base = c * 16 * _K0 + s * kc
        pltpu.sync_copy(gidx_hbm.at[pl.ds(base, kmax)], gidx_v)
        pltpu.sync_copy(sidx_hbm.at[pl.ds(base, kmax)], sidx_v)
        pltpu.sync_copy(zeros_hbm.at[pl.ds(s * rpt, rpt)],
                        agg_sh.at[pl.ds(s * rpt, rpt)])
        plsc.subcore_barrier()

        def start(j, buf, sem):
            pltpu.async_copy(table_hbm.at[gidx_v.at[j]], buf, sem)

        def finish(j, buf, sem):
            pltpu.make_async_copy(table_hbm.at[gidx_v.at[j]], buf, sem).wait()
            pltpu.sync_copy(buf, agg_sh.at[sidx_v.at[j]], add=True)

        start(0, v0, sem0)
        start(1, v1, sem1)

        def body(i, carry):
            j = i * 2
            finish(j, v0, sem0)
            start(j + 2, v0, sem0)
            finish(j + 1, v1, sem1)
            start(j + 3, v1, sem1)
            return carry

        lax.fori_loop(0, kc // 2 - 1, body, 0)
        finish(kc - 2, v0, sem0)
        finish(kc - 1, v1, sem1)
        plsc.subcore_barrier()
        pltpu.sync_copy(agg_sh.at[pl.ds(s * rpt, rpt)],
                        out_hbm.at[pl.ds(s * rpt, rpt), pl.ds(c * 16, 16)])

    return k(table, gidx, sidx, zeros)


def kernel(x, edge_index, is_reversed, Ws_st1, Wn_st1, b_st1,
           Ws_ts1, Wn_ts1, b_ts1, Ws_2, Wn_2, b_2):
    rev = is_reversed.astype(jnp.int32)
    pad = _EPAD - _E
    ei_pad = jnp.broadcast_to(jnp.array([[0], [_N]], jnp.int32), (2, pad))
    eip = jnp.concatenate([edge_index, ei_pad], axis=1).reshape(2, _ROWS, 128)
    revp = jnp.pad(rev, (0, pad)).reshape(_ROWS, 128)

    ys, yn, g1, s1, g2 = _tc_a(x, Ws_st1, Ws_ts1, Wn_st1, Wn_ts1, eip, revp)

    agg1 = _sc_segsum(yn.reshape(8 * _N, 16), g1, s1,
                      jnp.zeros((_AGG1, 16), jnp.float32), _AGG1)
    z = _tc_c(ys, agg1,
              b_st1.reshape(1, 16), b_ts1.reshape(1, 16),
              Ws_2[0:16], Ws_2[16:32], Wn_2[0:16], Wn_2[16:32])

    agg2 = _sc_segsum(z.reshape(8 * _N, 16), g2, eip[1],
                      jnp.zeros((_AGG2, 16), jnp.float32), _AGG2)
    return _tc_e(z, agg2, b_2.reshape(1, 16))


# speedup vs baseline: 1.3694x; 1.2068x over previous
import functools

import jax
import jax.numpy as jnp
from jax import lax
from jax.experimental import pallas as pl
from jax.experimental.pallas import tpu as pltpu
from jax.experimental.pallas import tpu_sc as plsc

_N = 10000
_E = 320000
_CH = 128
_K = 80
_K0 = 80
_K1 = 80
_NW = 32
_EPAD = _NW * _K * _CH
_ROWS = _EPAD // 128
_HALF = _N + 48
_AGG1 = 2 * _HALF
_AGG2 = _N + 112


def _tc_a_body(x_ref, wsst_ref, wsts_ref, wnst_ref, wnts_ref,
               ei_ref, rev_ref, ys_ref, yn_ref, g1_ref, s1_ref, g2_ref):
    x = x_ref[...]
    ys_ref[0:_N, 0:16] = jnp.dot(x, wsst_ref[...],
                                 preferred_element_type=jnp.float32)
    ys_ref[0:_N, 16:32] = jnp.dot(x, wsts_ref[...],
                                  preferred_element_type=jnp.float32)
    yn_ref[0:_N, 0:16] = jnp.dot(x, wnst_ref[...],
                                 preferred_element_type=jnp.float32)
    yn_ref[0:_N, 16:32] = jnp.dot(x, wnts_ref[...],
                                  preferred_element_type=jnp.float32)
    src = ei_ref[0]
    dst = ei_ref[1]
    rev = rev_ref[...]
    g1_ref[...] = src * 8 + rev
    s1_ref[...] = dst + _HALF * rev
    g2_ref[...] = src * 8 + 1


def _tc_a(x, wsst, wsts, wnst, wnts, eip, revp):
    return pl.pallas_call(
        _tc_a_body,
        out_shape=[
            jax.ShapeDtypeStruct((_N, 128), jnp.float32),
            jax.ShapeDtypeStruct((_N, 128), jnp.float32),
            jax.ShapeDtypeStruct((_ROWS, 128), jnp.int32),
            jax.ShapeDtypeStruct((_ROWS, 128), jnp.int32),
            jax.ShapeDtypeStruct((_ROWS, 128), jnp.int32),
        ],
    )(x, wsst, wsts, wnst, wnts, eip, revp)


def _tc_c_body(ys_ref, a_ref, bst_ref, bts_ref,
               ws2a_ref, ws2b_ref, wn2a_ref, wn2b_ref, z_ref):
    a_st = a_ref[0:_N, 0:16] + a_ref[0:_N, 16:32]
    a_ts = (a_ref[_HALF:_HALF + _N, 0:16]
            + a_ref[_HALF:_HALF + _N, 16:32])
    h_st = jnp.maximum(ys_ref[0:_N, 0:16] + a_st + bst_ref[...], 0.0)
    h_ts = jnp.maximum(ys_ref[0:_N, 16:32] + a_ts + bts_ref[...], 0.0)
    z_ref[0:_N, 0:16] = (
        jnp.dot(h_st, ws2a_ref[...], preferred_element_type=jnp.float32)
        + jnp.dot(h_ts, ws2b_ref[...], preferred_element_type=jnp.float32))
    z_ref[0:_N, 16:32] = (
        jnp.dot(h_st, wn2a_ref[...], preferred_element_type=jnp.float32)
        + jnp.dot(h_ts, wn2b_ref[...], preferred_element_type=jnp.float32))


def _tc_c(ys, agg1, bst, bts, ws2a, ws2b, wn2a, wn2b):
    return pl.pallas_call(
        _tc_c_body,
        out_shape=jax.ShapeDtypeStruct((_N, 128), jnp.float32),
    )(ys, agg1, bst, bts, ws2a, ws2b, wn2a, wn2b)


def _tc_e_body(z_ref, a_ref, b_ref, out_ref):
    h = (z_ref[0:_N, 0:16] + a_ref[0:_N, 0:16] + a_ref[0:_N, 16:32]
         + b_ref[...])
    m = jnp.max(h, axis=1, keepdims=True)
    e = jnp.exp(h - m)
    lse = m + jnp.log(jnp.sum(e, axis=1, keepdims=True))
    out_ref[...] = h - lse


def _tc_e(z, agg2, b2):
    return pl.pallas_call(
        _tc_e_body,
        out_shape=jax.ShapeDtypeStruct((_N, 16), jnp.float32),
    )(z, agg2, b2)


def _sc_segsum(table, gidx, sidx, zeros, agg_rows):
    rpt = agg_rows // 16
    kmax = max(_K0, _K1)
    mesh = plsc.VectorSubcoreMesh(core_axis_name="c", subcore_axis_name="s")

    @functools.partial(
        pl.kernel,
        out_type=jax.ShapeDtypeStruct((agg_rows, 128), jnp.float32),
        mesh=mesh,
        scratch_types=[
            pltpu.VMEM((kmax, _CH), jnp.int32),
            pltpu.VMEM((kmax, _CH), jnp.int32),
            pltpu.VMEM((_CH, 16), jnp.float32),
            pltpu.VMEM((_CH, 16), jnp.float32),
            pltpu.VMEM_SHARED((agg_rows, 16), jnp.float32),
            pltpu.SemaphoreType.DMA,
            pltpu.SemaphoreType.DMA,
        ],
        compiler_params=pltpu.CompilerParams(use_tc_tiling_on_sc=False),
    )
    def k(table_hbm, gidx_hbm, sidx_hbm, zeros_hbm, out_hbm,
          gidx_v, sidx_v, v0, v1, agg_sh, sem0, sem1):
        c = lax.axis_index("c")
        s = lax.axis_index("s")
        kc = _K0 + c * (_K1 - _K0)
        base = c * 16 * _K0 + s * kc
        pltpu.sync_copy(gidx_hbm.at[pl.ds(base, kmax)], gidx_v)
        pltpu.sync_copy(sidx_hbm.at[pl.ds(base, kmax)], sidx_v)
        pltpu.sync_copy(zeros_hbm.at[pl.ds(s * rpt, rpt)],
                        agg_sh.at[pl.ds(s * rpt, rpt)])
        plsc.subcore_barrier()

        def start(j, buf, sem):
            pltpu.async_copy(table_hbm.at[gidx_v.at[j]], buf, sem)

        def finish(j, buf, sem):
            pltpu.make_async_copy(table_hbm.at[gidx_v.at[j]], buf, sem).wait()
            pltpu.sync_copy(buf, agg_sh.at[sidx_v.at[j]], add=True)

        start(0, v0, sem0)
        start(1, v1, sem1)

        def body(i, carry):
            j = i * 2
            finish(j, v0, sem0)
            start(j + 2, v0, sem0)
            finish(j + 1, v1, sem1)
            start(j + 3, v1, sem1)
            return carry

        lax.fori_loop(0, kc // 2 - 1, body, 0)
        finish(kc - 2, v0, sem0)
        finish(kc - 1, v1, sem1)
        plsc.subcore_barrier()
        pltpu.sync_copy(agg_sh.at[pl.ds(s * rpt, rpt)],
                        out_hbm.at[pl.ds(s * rpt, rpt), pl.ds(c * 16, 16)])

    return k(table, gidx, sidx, zeros)


def kernel(x, edge_index, is_reversed, Ws_st1, Wn_st1, b_st1,
           Ws_ts1, Wn_ts1, b_ts1, Ws_2, Wn_2, b_2):
    rev = is_reversed.astype(jnp.int32)
    pad = _EPAD - _E
    pad_i = jnp.arange(pad, dtype=jnp.int32)
    ei_pad = jnp.stack([pad_i % _N, _N + (pad_i % 48)])
    eip = jnp.concatenate([edge_index, ei_pad], axis=1).reshape(2, _ROWS, 128)
    revp = jnp.pad(rev, (0, pad)).reshape(_ROWS, 128)

    ys, yn, g1, s1, g2 = _tc_a(x, Ws_st1, Ws_ts1, Wn_st1, Wn_ts1, eip, revp)

    agg1 = _sc_segsum(yn.reshape(8 * _N, 16), g1, s1,
                      jnp.zeros((_AGG1, 16), jnp.float32), _AGG1)
    z = _tc_c(ys, agg1,
              b_st1.reshape(1, 16), b_ts1.reshape(1, 16),
              Ws_2[0:16], Ws_2[16:32], Wn_2[0:16], Wn_2[16:32])

    agg2 = _sc_segsum(z.reshape(8 * _N, 16), g2, eip[1],
                      jnp.zeros((_AGG2, 16), jnp.float32), _AGG2)
    return _tc_e(z, agg2, b_2.reshape(1, 16))


# speedup vs baseline: 1.3901x; 1.0151x over previous
import functools

import jax
import jax.numpy as jnp
from jax import lax
from jax.experimental import pallas as pl
from jax.experimental.pallas import tpu as pltpu
from jax.experimental.pallas import tpu_sc as plsc

_N = 10000
_E = 320000
_CH = 128
_K = 80
_K0 = 80
_K1 = 80
_NW = 32
_EPAD = _NW * _K * _CH
_ROWS = _EPAD // 128
_HALF = _N + 48
_AGG1 = 2 * _HALF
_AGG2 = _N + 112


def _tc_a_body(x_ref, w1_ref, ei_ref, rev_ref, yn_ref, g1_ref, s1_ref, g2_ref):
    yn_ref[0:_N, 0:64] = jnp.dot(x_ref[...], w1_ref[...],
                                 preferred_element_type=jnp.float32)
    src = ei_ref[0]
    dst = ei_ref[1]
    rev = rev_ref[...]
    g1_ref[...] = src * 8 + rev
    s1_ref[...] = dst + _HALF * rev
    g2_ref[...] = src * 8 + 1


def _tc_a(x, w1, eip, revp):
    return pl.pallas_call(
        _tc_a_body,
        out_shape=[
            jax.ShapeDtypeStruct((_N, 128), jnp.float32),
            jax.ShapeDtypeStruct((_ROWS, 128), jnp.int32),
            jax.ShapeDtypeStruct((_ROWS, 128), jnp.int32),
            jax.ShapeDtypeStruct((_ROWS, 128), jnp.int32),
        ],
    )(x, w1, eip, revp)


def _tc_c_body(yn_ref, a_ref, bst_ref, bts_ref, ws2_ref, wn2_ref, z_ref):
    a_st = a_ref[0:_N, 0:16] + a_ref[0:_N, 16:32]
    a_ts = (a_ref[_HALF:_HALF + _N, 0:16]
            + a_ref[_HALF:_HALF + _N, 16:32])
    h_st = jnp.maximum(yn_ref[0:_N, 32:48] + a_st + bst_ref[...], 0.0)
    h_ts = jnp.maximum(yn_ref[0:_N, 48:64] + a_ts + bts_ref[...], 0.0)
    ws2 = ws2_ref[...]
    wn2 = wn2_ref[...]
    z_ref[0:_N, 0:16] = (
        jnp.dot(h_st, ws2[0:16], preferred_element_type=jnp.float32)
        + jnp.dot(h_ts, ws2[16:32], preferred_element_type=jnp.float32))
    z_ref[0:_N, 16:32] = (
        jnp.dot(h_st, wn2[0:16], preferred_element_type=jnp.float32)
        + jnp.dot(h_ts, wn2[16:32], preferred_element_type=jnp.float32))


def _tc_c(yn, agg1, bst, bts, ws2, wn2):
    return pl.pallas_call(
        _tc_c_body,
        out_shape=jax.ShapeDtypeStruct((_N, 128), jnp.float32),
    )(yn, agg1, bst, bts, ws2, wn2)


def _tc_e_body(z_ref, a_ref, b_ref, out_ref):
    h = (z_ref[0:_N, 0:16] + a_ref[0:_N, 0:16] + a_ref[0:_N, 16:32]
         + b_ref[...])
    m = jnp.max(h, axis=1, keepdims=True)
    e = jnp.exp(h - m)
    lse = m + jnp.log(jnp.sum(e, axis=1, keepdims=True))
    out_ref[...] = h - lse


def _tc_e(z, agg2, b2):
    return pl.pallas_call(
        _tc_e_body,
        out_shape=jax.ShapeDtypeStruct((_N, 16), jnp.float32),
    )(z, agg2, b2)


def _sc_segsum(table, gidx, sidx, zeros, agg_rows):
    rpt = agg_rows // 16
    kmax = max(_K0, _K1)
    mesh = plsc.VectorSubcoreMesh(core_axis_name="c", subcore_axis_name="s")

    @functools.partial(
        pl.kernel,
        out_type=jax.ShapeDtypeStruct((agg_rows, 128), jnp.float32),
        mesh=mesh,
        scratch_types=[
            pltpu.VMEM((kmax, _CH), jnp.int32),
            pltpu.VMEM((kmax, _CH), jnp.int32),
            pltpu.VMEM((_CH, 16), jnp.float32),
            pltpu.VMEM((_CH, 16), jnp.float32),
            pltpu.VMEM_SHARED((agg_rows, 16), jnp.float32),
            pltpu.SemaphoreType.DMA,
            pltpu.SemaphoreType.DMA,
        ],
        compiler_params=pltpu.CompilerParams(use_tc_tiling_on_sc=False),
    )
    def k(table_hbm, gidx_hbm, sidx_hbm, zeros_hbm, out_hbm,
          gidx_v, sidx_v, v0, v1, agg_sh, sem0, sem1):
        c = lax.axis_index("c")
        s = lax.axis_index("s")
        kc = _K0 + c * (_K1 - _K0)
        base = c * 16 * _K0 + s * kc
        pltpu.sync_copy(gidx_hbm.at[pl.ds(base, kmax)], gidx_v)
        pltpu.sync_copy(sidx_hbm.at[pl.ds(base, kmax)], sidx_v)
        pltpu.sync_copy(zeros_hbm.at[pl.ds(s * rpt, rpt)],
                        agg_sh.at[pl.ds(s * rpt, rpt)])
        plsc.subcore_barrier()

        def start(j, buf, sem):
            pltpu.async_copy(table_hbm.at[gidx_v.at[j]], buf, sem)

        def finish(j, buf, sem):
            pltpu.make_async_copy(table_hbm.at[gidx_v.at[j]], buf, sem).wait()
            pltpu.sync_copy(buf, agg_sh.at[sidx_v.at[j]], add=True)

        start(0, v0, sem0)
        start(1, v1, sem1)

        def body(i, carry):
            j = i * 2
            finish(j, v0, sem0)
            start(j + 2, v0, sem0)
            finish(j + 1, v1, sem1)
            start(j + 3, v1, sem1)
            return carry

        lax.fori_loop(0, kc // 2 - 1, body, 0)
        finish(kc - 2, v0, sem0)
        finish(kc - 1, v1, sem1)
        plsc.subcore_barrier()
        pltpu.sync_copy(agg_sh.at[pl.ds(s * rpt, rpt)],
                        out_hbm.at[pl.ds(s * rpt, rpt), pl.ds(c * 16, 16)])

    return k(table, gidx, sidx, zeros)


def kernel(x, edge_index, is_reversed, Ws_st1, Wn_st1, b_st1,
           Ws_ts1, Wn_ts1, b_ts1, Ws_2, Wn_2, b_2):
    rev = is_reversed.astype(jnp.int32)
    pad = _EPAD - _E
    pad_i = jnp.arange(pad, dtype=jnp.int32)
    ei_pad = jnp.stack([pad_i % _N, _N + (pad_i % 48)])
    eip = jnp.concatenate([edge_index, ei_pad], axis=1).reshape(2, _ROWS, 128)
    revp = jnp.pad(rev, (0, pad)).reshape(_ROWS, 128)

    w1 = jnp.concatenate([Wn_st1, Wn_ts1, Ws_st1, Ws_ts1], axis=1)
    yn, g1, s1, g2 = _tc_a(x, w1, eip, revp)

    agg1 = _sc_segsum(yn.reshape(8 * _N, 16), g1, s1,
                      jnp.zeros((_AGG1, 16), jnp.float32), _AGG1)
    z = _tc_c(yn, agg1,
              b_st1.reshape(1, 16), b_ts1.reshape(1, 16), Ws_2, Wn_2)

    agg2 = _sc_segsum(z.reshape(8 * _N, 16), g2, eip[1],
                      jnp.zeros((_AGG2, 16), jnp.float32), _AGG2)
    return _tc_e(z, agg2, b_2.reshape(1, 16))
